# Initial kernel scaffold; baseline (speedup 1.0000x reference)
import jax
import jax.numpy as jnp
from jax.experimental import pallas as pl


def kernel(x, edge_index, etypes, norm, weight, h_bias):
    raise NotImplementedError("write your pallas kernel here")


# speedup vs baseline: 15.0486x; 15.0486x over previous
import functools

import jax
import jax.numpy as jnp
from jax import lax
from jax.experimental import pallas as pl
from jax.experimental.pallas import tpu as pltpu
from jax.experimental.pallas import tpu_sc as plsc

N = 10000
E = 320000
D = 128
R = 8

NC = 2
NS = 16
NW = NC * NS
L = 16

C = 128
NCHUNK = E // C
ROWS_PER_TILE = 624
TAIL_ROW0 = NS * ROWS_PER_TILE
TAIL_ROWS = N - TAIL_ROW0

_BN = 1000


def _proj_body(x_ref, w_ref, out_ref):
    xb = x_ref[...]
    for r in range(R):
        out_ref[:, r, :] = jnp.dot(xb, w_ref[r], preferred_element_type=jnp.float32)


def _proj(x, weight):
    return pl.pallas_call(
        _proj_body,
        grid=(N // _BN,),
        in_specs=[
            pl.BlockSpec((_BN, D), lambda i: (i, 0)),
            pl.BlockSpec((R, D, D), lambda i: (0, 0, 0)),
        ],
        out_specs=pl.BlockSpec((_BN, R, D), lambda i: (i, 0, 0)),
        out_shape=jax.ShapeDtypeStruct((N, R, D), jnp.float32),
    )(x, weight)


def _combine_body(p_ref, b_ref, out_ref):
    out_ref[...] = p_ref[0] + p_ref[1] + b_ref[...]


def _combine(partials, h_bias):
    return pl.pallas_call(
        _combine_body,
        grid=(N // _BN,),
        in_specs=[
            pl.BlockSpec((NC, _BN, D), lambda i: (0, i, 0)),
            pl.BlockSpec((1, D), lambda i: (0, 0)),
        ],
        out_specs=pl.BlockSpec((_BN, D), lambda i: (i, 0)),
        out_shape=jax.ShapeDtypeStruct((N, D), jnp.float32),
    )(partials, h_bias.reshape(1, D))


@functools.cache
def _build_edge_kernel():
    mesh = plsc.VectorSubcoreMesh(core_axis_name="c", subcore_axis_name="s")
    return functools.partial(
        pl.kernel,
        mesh=mesh,
        out_type=jax.ShapeDtypeStruct((NC, N, D), jnp.float32),
        scratch_types=[
            pltpu.VMEM((C,), jnp.int32),
            pltpu.VMEM((C,), jnp.int32),
            pltpu.VMEM((C,), jnp.int32),
            pltpu.VMEM((C,), jnp.float32),
            pltpu.VMEM((C,), jnp.int32),
            pltpu.VMEM((C, D), jnp.float32),
            pltpu.VMEM_SHARED((N, D), jnp.float32),
            pltpu.SemaphoreType.DMA,
        ],
    )(_edge_body)


def _edge_body(src_hbm, et_hbm, dst_hbm, norm_hbm, proj_hbm, out_hbm,
               src_v, et_v, dst_v, norm_v, idx_v, rows_v, acc, sem):
    c = lax.axis_index("c")
    s = lax.axis_index("s")
    w = s * NC + c

    def _zero_rows(e, _):
        for k in range(D // L):
            rows_v[e, pl.ds(k * L, L)] = jnp.zeros((L,), jnp.float32)
        return 0
    lax.fori_loop(0, C, _zero_rows, 0)
    row0 = s * ROWS_PER_TILE
    for j in range(4):
        pltpu.sync_copy(rows_v, acc.at[pl.ds(row0 + j * C, C)])
    pltpu.sync_copy(rows_v.at[pl.ds(0, ROWS_PER_TILE - 4 * C)],
                    acc.at[pl.ds(row0 + 4 * C, ROWS_PER_TILE - 4 * C)])

    @pl.when(s == NS - 1)
    def _zero_tail():
        pltpu.sync_copy(rows_v.at[pl.ds(0, TAIL_ROWS)],
                        acc.at[pl.ds(TAIL_ROW0, TAIL_ROWS)])
    plsc.subcore_barrier()

    n_chunks = 78 + jnp.where(w < NCHUNK - 78 * NW, 1, 0)

    def _chunk_body(t, _):
        base = (t * NW + w) * C
        pltpu.sync_copy(src_hbm.at[pl.ds(base, C)], src_v)
        pltpu.sync_copy(et_hbm.at[pl.ds(base, C)], et_v)
        pltpu.sync_copy(dst_hbm.at[pl.ds(base, C)], dst_v)
        pltpu.sync_copy(norm_hbm.at[pl.ds(base, C)], norm_v)
        for g in range(C // L):
            sl = pl.ds(g * L, L)
            idx_v[sl] = src_v[sl] * R + et_v[sl]
        pltpu.async_copy(proj_hbm.at[idx_v], rows_v, sem).wait()

        def _scale(g, _):
            nv = norm_v[pl.ds(g * L, L)]
            for l in range(L):
                sv = nv[l]
                e = g * L + l
                for k in range(D // L):
                    sk = pl.ds(k * L, L)
                    rows_v[e, sk] = rows_v[e, sk] * sv
            return 0
        lax.fori_loop(0, C // L, _scale, 0)

        pltpu.sync_copy(rows_v, acc.at[dst_v], add=True)
        return 0

    lax.fori_loop(0, n_chunks, _chunk_body, 0)
    plsc.subcore_barrier()

    pltpu.sync_copy(acc.at[pl.ds(row0, ROWS_PER_TILE)],
                    out_hbm.at[c, pl.ds(row0, ROWS_PER_TILE)])

    @pl.when(s == NS - 1)
    def _copy_tail():
        pltpu.sync_copy(acc.at[pl.ds(TAIL_ROW0, TAIL_ROWS)],
                        out_hbm.at[c, pl.ds(TAIL_ROW0, TAIL_ROWS)])


def kernel(x, edge_index, etypes, norm, weight, h_bias):
    proj = _proj(x, weight).reshape(N * R, D)
    src = edge_index[0]
    dst = edge_index[1]
    partials = _build_edge_kernel()(src, etypes, dst, norm.reshape(E), proj)
    return _combine(partials, h_bias)


# speedup vs baseline: 29.8666x; 1.9847x over previous
import functools

import jax
import jax.numpy as jnp
from jax import lax
from jax.experimental import pallas as pl
from jax.experimental.pallas import tpu as pltpu
from jax.experimental.pallas import tpu_sc as plsc

N = 10000
E = 320000
D = 128
R = 8

NC = 2
NS = 16
NW = NC * NS
L = 16

C = 80
NCHUNK = E // C
CH_PER_W = NCHUNK // NW
EPW = CH_PER_W * C

ROWS_PER_TILE = 624
TAIL_ROW0 = NS * ROWS_PER_TILE
TAIL_ROWS = N - TAIL_ROW0

_BN = 1000


def _proj_body(x_ref, w_ref, out_ref):
    xb = x_ref[...]
    for r in range(R):
        out_ref[:, r, :] = jnp.dot(xb, w_ref[r], preferred_element_type=jnp.float32)


def _proj(x, weight):
    return pl.pallas_call(
        _proj_body,
        grid=(N // _BN,),
        in_specs=[
            pl.BlockSpec((_BN, D), lambda i: (i, 0)),
            pl.BlockSpec((R, D, D), lambda i: (0, 0, 0)),
        ],
        out_specs=pl.BlockSpec((_BN, R, D), lambda i: (i, 0, 0)),
        out_shape=jax.ShapeDtypeStruct((N, R, D), jnp.float32),
    )(x, weight)


DST_BITS = 14


def _pack_body(src_ref, et_ref, dst_ref, out_ref):
    idx = src_ref[...] * R + et_ref[...]
    out_ref[...] = jnp.left_shift(idx, DST_BITS) + dst_ref[...]


def _pack_meta(src, etypes, dst):
    e2 = E // 128
    out = pl.pallas_call(
        _pack_body,
        in_specs=[pl.BlockSpec((e2, 128), lambda: (0, 0)),
                  pl.BlockSpec((e2, 128), lambda: (0, 0)),
                  pl.BlockSpec((e2, 128), lambda: (0, 0))],
        out_specs=pl.BlockSpec((e2, 128), lambda: (0, 0)),
        out_shape=jax.ShapeDtypeStruct((e2, 128), jnp.int32),
    )(src.reshape(e2, 128), etypes.reshape(e2, 128), dst.reshape(e2, 128))
    return out.reshape(E)


def _combine_body(p_ref, b_ref, out_ref):
    out_ref[...] = p_ref[0] + p_ref[1] + b_ref[...]


def _combine(partials, h_bias):
    return pl.pallas_call(
        _combine_body,
        grid=(N // _BN,),
        in_specs=[
            pl.BlockSpec((NC, _BN, D), lambda i: (0, i, 0)),
            pl.BlockSpec((1, D), lambda i: (0, 0)),
        ],
        out_specs=pl.BlockSpec((_BN, D), lambda i: (i, 0)),
        out_shape=jax.ShapeDtypeStruct((N, D), jnp.float32),
    )(partials, h_bias.reshape(1, D))


@functools.cache
def _build_edge_kernel():
    mesh = plsc.VectorSubcoreMesh(core_axis_name="c", subcore_axis_name="s")
    return functools.partial(
        pl.kernel,
        mesh=mesh,
        out_type=jax.ShapeDtypeStruct((NC, N, D), jnp.float32),
        scratch_types=[
            pltpu.VMEM((EPW,), jnp.int32),
            pltpu.VMEM((EPW,), jnp.float32),
            pltpu.VMEM((2, C), jnp.int32),
            pltpu.VMEM((2, C), jnp.int32),
            pltpu.VMEM((C, D), jnp.float32),
            pltpu.VMEM((C, D), jnp.float32),
            pltpu.VMEM_SHARED((N, D), jnp.float32),
            pltpu.SemaphoreType.DMA((2,)),
        ],
    )(_edge_body)


def _edge_body(packed_hbm, norm_hbm, proj_hbm, out_hbm,
               packed_v, norm_v, idxc, dstc, rows_a, rows_b, acc, gsem):
    c = lax.axis_index("c")
    s = lax.axis_index("s")
    w = s * NC + c

    def _zero_rows(e, _):
        for k in range(D // L):
            rows_a[e, pl.ds(k * L, L)] = jnp.zeros((L,), jnp.float32)
        return 0
    lax.fori_loop(0, C, _zero_rows, 0)
    row0 = s * ROWS_PER_TILE
    for j in range(ROWS_PER_TILE // C):
        pltpu.sync_copy(rows_a, acc.at[pl.ds(row0 + j * C, C)])
    rem = ROWS_PER_TILE - (ROWS_PER_TILE // C) * C
    if rem:
        pltpu.sync_copy(rows_a.at[pl.ds(0, rem)],
                        acc.at[pl.ds(row0 + ROWS_PER_TILE - rem, rem)])

    @pl.when(s == NS - 1)
    def _zero_tail():
        pltpu.sync_copy(rows_a.at[pl.ds(0, TAIL_ROWS)],
                        acc.at[pl.ds(TAIL_ROW0, TAIL_ROWS)])
    plsc.subcore_barrier()

    ebase = w * EPW
    pltpu.sync_copy(packed_hbm.at[pl.ds(ebase, EPW)], packed_v)
    pltpu.sync_copy(norm_hbm.at[pl.ds(ebase, EPW)], norm_v)

    def _unpack(t, slot):
        for g in range(C // L):
            pv = packed_v[pl.ds(t * C + g * L, L)]
            idxc[slot, pl.ds(g * L, L)] = jax.lax.shift_right_logical(
                pv, DST_BITS)
            dstc[slot, pl.ds(g * L, L)] = jnp.bitwise_and(
                pv, (1 << DST_BITS) - 1)

    def _gather(t, buf, sem_i):
        _unpack(t, sem_i)
        pltpu.async_copy(proj_hbm.at[idxc.at[sem_i]], buf, gsem.at[sem_i])

    def _gather_wait(t, buf, sem_i):
        pltpu.make_async_copy(proj_hbm.at[idxc.at[sem_i]], buf,
                              gsem.at[sem_i]).wait()

    def _scale(buf, t):
        def body(g, _):
            nv = norm_v[pl.ds(t * C + g * L, L)]
            for l in range(L):
                sv = nv[l]
                e = g * L + l
                for k in range(D // L):
                    sk = pl.ds(k * L, L)
                    buf[e, sk] = buf[e, sk] * sv
            return 0
        lax.fori_loop(0, C // L, body, 0)

    def _scatter(buf, slot):
        pltpu.sync_copy(buf, acc.at[dstc.at[slot]], add=True)

    _gather(0, rows_a, 0)

    def _pipe(t2, _):
        ta = 2 * t2
        tb = ta + 1

        @pl.when(tb < CH_PER_W)
        def _prefetch_b():
            _gather(tb, rows_b, 1)
        _gather_wait(ta, rows_a, 0)
        _scale(rows_a, ta)
        _scatter(rows_a, 0)

        @pl.when(tb < CH_PER_W)
        def _do_b():
            @pl.when(tb + 1 < CH_PER_W)
            def _prefetch_a():
                _gather(tb + 1, rows_a, 0)
            _gather_wait(tb, rows_b, 1)
            _scale(rows_b, tb)
            _scatter(rows_b, 1)
        return 0

    lax.fori_loop(0, (CH_PER_W + 1) // 2, _pipe, 0)

    plsc.subcore_barrier()

    pltpu.sync_copy(acc.at[pl.ds(row0, ROWS_PER_TILE)],
                    out_hbm.at[c, pl.ds(row0, ROWS_PER_TILE)])

    @pl.when(s == NS - 1)
    def _copy_tail():
        pltpu.sync_copy(acc.at[pl.ds(TAIL_ROW0, TAIL_ROWS)],
                        out_hbm.at[c, pl.ds(TAIL_ROW0, TAIL_ROWS)])


def kernel(x, edge_index, etypes, norm, weight, h_bias):
    proj = _proj(x, weight).reshape(N * R, D)
    packed = _pack_meta(edge_index[0], etypes, edge_index[1])
    partials = _build_edge_kernel()(packed, norm.reshape(E), proj)
    return _combine(partials, h_bias)


# speedup vs baseline: 33.7185x; 1.1290x over previous
import functools

import jax
import jax.numpy as jnp
from jax import lax
from jax.experimental import pallas as pl
from jax.experimental.pallas import tpu as pltpu
from jax.experimental.pallas import tpu_sc as plsc

N = 10000
E = 320000
D = 128
R = 8

NC = 2
NS = 16
NW = NC * NS
L = 16

C = 80
NCHUNK = E // C
CH_PER_W = NCHUNK // NW
EPW = CH_PER_W * C

ROWS_PER_TILE = 624
TAIL_ROW0 = NS * ROWS_PER_TILE
TAIL_ROWS = N - TAIL_ROW0

_BN = 1000


def _proj_body(x_ref, w_ref, out_ref):
    xb = x_ref[...]
    for r in range(R):
        out_ref[:, r, :] = jnp.dot(xb, w_ref[r], preferred_element_type=jnp.float32)


def _proj(x, weight):
    return pl.pallas_call(
        _proj_body,
        grid=(N // _BN,),
        in_specs=[
            pl.BlockSpec((_BN, D), lambda i: (i, 0)),
            pl.BlockSpec((R, D, D), lambda i: (0, 0, 0)),
        ],
        out_specs=pl.BlockSpec((_BN, R, D), lambda i: (i, 0, 0)),
        out_shape=jax.ShapeDtypeStruct((N, R, D), jnp.float32),
    )(x, weight)


DST_BITS = 14


def _pack_body(src_ref, et_ref, dst_ref, out_ref):
    idx = src_ref[...] * R + et_ref[...]
    out_ref[...] = jnp.left_shift(idx, DST_BITS) + dst_ref[...]


def _pack_meta(src, etypes, dst):
    e2 = E // 128
    out = pl.pallas_call(
        _pack_body,
        in_specs=[pl.BlockSpec((e2, 128), lambda: (0, 0)),
                  pl.BlockSpec((e2, 128), lambda: (0, 0)),
                  pl.BlockSpec((e2, 128), lambda: (0, 0))],
        out_specs=pl.BlockSpec((e2, 128), lambda: (0, 0)),
        out_shape=jax.ShapeDtypeStruct((e2, 128), jnp.int32),
    )(src.reshape(e2, 128), etypes.reshape(e2, 128), dst.reshape(e2, 128))
    return out.reshape(E)


def _combine_body(p_ref, b_ref, out_ref):
    out_ref[...] = p_ref[0] + p_ref[1] + b_ref[...]


def _combine(partials, h_bias):
    return pl.pallas_call(
        _combine_body,
        grid=(N // _BN,),
        in_specs=[
            pl.BlockSpec((NC, _BN, D), lambda i: (0, i, 0)),
            pl.BlockSpec((1, D), lambda i: (0, 0)),
        ],
        out_specs=pl.BlockSpec((_BN, D), lambda i: (i, 0)),
        out_shape=jax.ShapeDtypeStruct((N, D), jnp.float32),
    )(partials, h_bias.reshape(1, D))


@functools.cache
def _build_edge_kernel():
    mesh = plsc.VectorSubcoreMesh(core_axis_name="c", subcore_axis_name="s")
    return functools.partial(
        pl.kernel,
        mesh=mesh,
        out_type=jax.ShapeDtypeStruct((NC, N, D), jnp.float32),
        scratch_types=[
            pltpu.VMEM((4, C), jnp.int32),
            pltpu.VMEM((4, C), jnp.float32),
            pltpu.VMEM((4, C), jnp.int32),
            pltpu.VMEM((4, C), jnp.int32),
            pltpu.VMEM((C, D), jnp.float32),
            pltpu.VMEM((C, D), jnp.float32),
            pltpu.VMEM((C, D), jnp.float32),
            pltpu.VMEM((C, D), jnp.float32),
            pltpu.VMEM_SHARED((N, D), jnp.float32),
            pltpu.SemaphoreType.DMA((4,)),
            pltpu.SemaphoreType.DMA((4,)),
            pltpu.SemaphoreType.DMA((4,)),
            pltpu.SemaphoreType.DMA((4,)),
        ],
    )(_edge_body)


def _edge_body(packed_hbm, norm_hbm, proj_hbm, out_hbm,
               packedc, normc, idxc, dstc, r0, r1, r2, r3, acc,
               pms, nms, gsem, ssem):
    c = lax.axis_index("c")
    s = lax.axis_index("s")
    w = s * NC + c
    rows = (r0, r1, r2, r3)

    def _zero_rows(e, _):
        for k in range(D // L):
            r0[e, pl.ds(k * L, L)] = jnp.zeros((L,), jnp.float32)
        return 0
    lax.fori_loop(0, C, _zero_rows, 0)
    row0 = s * ROWS_PER_TILE
    for j in range(ROWS_PER_TILE // C):
        pltpu.sync_copy(r0, acc.at[pl.ds(row0 + j * C, C)])
    rem = ROWS_PER_TILE - (ROWS_PER_TILE // C) * C
    if rem:
        pltpu.sync_copy(r0.at[pl.ds(0, rem)],
                        acc.at[pl.ds(row0 + ROWS_PER_TILE - rem, rem)])

    @pl.when(s == NS - 1)
    def _zero_tail():
        pltpu.sync_copy(r0.at[pl.ds(0, TAIL_ROWS)],
                        acc.at[pl.ds(TAIL_ROW0, TAIL_ROWS)])
    plsc.subcore_barrier()

    ebase = w * EPW

    def _meta_issue(t, slot):
        sl = pl.ds(ebase + t * C, C)
        pltpu.async_copy(packed_hbm.at[sl], packedc.at[slot], pms.at[slot])
        pltpu.async_copy(norm_hbm.at[sl], normc.at[slot], nms.at[slot])

    def _meta_wait(t, slot):
        sl = pl.ds(ebase + t * C, C)
        pltpu.make_async_copy(packed_hbm.at[sl], packedc.at[slot],
                              pms.at[slot]).wait()
        pltpu.make_async_copy(norm_hbm.at[sl], normc.at[slot],
                              nms.at[slot]).wait()

    def _unpack(slot):
        for g in range(C // L):
            sl = pl.ds(g * L, L)
            pv = packedc[slot, sl]
            idxc[slot, sl] = jax.lax.shift_right_logical(pv, DST_BITS)
            dstc[slot, sl] = jnp.bitwise_and(pv, (1 << DST_BITS) - 1)

    def _gather_issue(slot):
        pltpu.async_copy(proj_hbm.at[idxc.at[slot]], rows[slot],
                         gsem.at[slot])

    def _gather_wait(slot):
        pltpu.make_async_copy(proj_hbm.at[idxc.at[slot]], rows[slot],
                              gsem.at[slot]).wait()

    def _scale(slot):
        buf = rows[slot]

        def body(g, _):
            nv = normc[slot, pl.ds(g * L, L)]
            for l in range(L):
                sv = nv[l]
                e = g * L + l
                for k in range(D // L):
                    sk = pl.ds(k * L, L)
                    buf[e, sk] = buf[e, sk] * sv
            return 0
        lax.fori_loop(0, C // L, body, 0)

    def _scatter_issue(slot):
        pltpu.async_copy(rows[slot], acc.at[dstc.at[slot]], ssem.at[slot],
                         add=True)

    def _scatter_wait(slot):
        pltpu.make_async_copy(rows[slot], acc.at[dstc.at[slot]],
                              ssem.at[slot]).wait()

    for t in range(4):
        _meta_issue(t, t)
    for t in range(2):
        _meta_wait(t, t)
        _unpack(t)
        _gather_issue(t)

    def _step(t4, _):
        for i in range(4):
            t = 4 * t4 + i
            j = i
            j2 = (i + 2) % 4

            @pl.when(t < CH_PER_W)
            def _process():
                @pl.when(t + 2 < CH_PER_W)
                def _prefetch():
                    @pl.when(t >= 2)
                    def _drain_prev():
                        _scatter_wait(j2)
                    _meta_wait(t + 2, j2)
                    _unpack(j2)
                    _gather_issue(j2)
                _gather_wait(j)
                _scale(j)

                @pl.when(t + 4 < CH_PER_W)
                def _meta_next():
                    _meta_issue(t + 4, j)
                _scatter_issue(j)
        return 0

    lax.fori_loop(0, (CH_PER_W + 3) // 4, _step, 0)

    for j in (1, 2, 3, 0):
        _scatter_wait(j)

    plsc.subcore_barrier()

    pltpu.sync_copy(acc.at[pl.ds(row0, ROWS_PER_TILE)],
                    out_hbm.at[c, pl.ds(row0, ROWS_PER_TILE)])

    @pl.when(s == NS - 1)
    def _copy_tail():
        pltpu.sync_copy(acc.at[pl.ds(TAIL_ROW0, TAIL_ROWS)],
                        out_hbm.at[c, pl.ds(TAIL_ROW0, TAIL_ROWS)])


def kernel(x, edge_index, etypes, norm, weight, h_bias):
    proj = _proj(x, weight).reshape(N * R, D)
    packed = _pack_meta(edge_index[0], etypes, edge_index[1])
    partials = _build_edge_kernel()(packed, norm.reshape(E), proj)
    return _combine(partials, h_bias)
